# baseline (speedup 1.0000x reference)
import functools

import jax
import jax.numpy as jnp
import numpy as np
from jax import lax
from jax.experimental import pallas as pl
from jax.experimental.pallas import tpu as pltpu
from jax.experimental.pallas import tpu_sc as plsc

_STRIDE = 16
_FH = _FW = 48
_C = 512
_HID = 2048
_IMG = float(_FH * _STRIDE)
_PRE = 2000
_POST = 300
_IOU = 0.7
_NA = _FH * _FW * 9
_NPAD = 21504
_ROWS = _NPAD // 128
_NSUB = 16
_PERW = _NPAD // _NSUB
_CCAP = 2176
_CROWS = _CCAP // 128


def _make_anchors(h, w):
    sizes = np.array([128.0, 256.0, 512.0])
    ratios = np.array([0.5, 1.0, 2.0])
    ws = (sizes[None, :] / np.sqrt(ratios)[:, None]).reshape(-1)
    hs = (sizes[None, :] * np.sqrt(ratios)[:, None]).reshape(-1)
    cx = (np.arange(w) + 0.5) * _STRIDE
    cy = (np.arange(h) + 0.5) * _STRIDE
    cxg, cyg = np.meshgrid(cx, cy)
    cxg = np.broadcast_to(cxg[..., None], (h, w, 9))
    cyg = np.broadcast_to(cyg[..., None], (h, w, 9))
    aw = np.broadcast_to(ws, (h, w, 9))
    ah = np.broadcast_to(hs, (h, w, 9))
    return np.stack([cxg, cyg, aw, ah], axis=-1).reshape(-1, 4).astype(np.float32)


_ANCHORS = _make_anchors(_FH, _FW)

_APLANES = [
    np.concatenate([_ANCHORS[:, j],
                    np.ones(_NPAD - _NA, np.float32)]).reshape(_ROWS, 128)
    for j in range(4)
]


def _conv_body(x_ref, w_ref, b_ref, h_ref):
    acc = jax.lax.dot_general(
        x_ref[...], w_ref[...], (((1,), (0,)), ((), ())),
        preferred_element_type=jnp.float32)
    h_ref[...] = jnp.maximum(acc + b_ref[...], 0.0)


def _heads_body(h_ref, wh_ref, bh_ref, l_ref):
    acc = jax.lax.dot_general(
        h_ref[...], wh_ref[...], (((1,), (0,)), ((), ())),
        preferred_element_type=jnp.float32)
    l_ref[...] = acc + bh_ref[...]


def _conv_heads(x, w, b1, wh, bh):
    m_blk, n_blk = 256, 1024
    hidden = pl.pallas_call(
        _conv_body,
        grid=(_HID // n_blk, 2304 // m_blk),
        in_specs=[
            pl.BlockSpec((m_blk, 4608), lambda n, m: (m, 0)),
            pl.BlockSpec((4608, n_blk), lambda n, m: (0, n)),
            pl.BlockSpec((1, n_blk), lambda n, m: (0, n)),
        ],
        out_specs=pl.BlockSpec((m_blk, n_blk), lambda n, m: (m, n)),
        out_shape=jax.ShapeDtypeStruct((2304, _HID), jnp.float32),
        compiler_params=pltpu.CompilerParams(
            dimension_semantics=("arbitrary", "arbitrary")),
    )(x, w, b1)
    logits = pl.pallas_call(
        _heads_body,
        grid=(2304 // m_blk,),
        in_specs=[
            pl.BlockSpec((m_blk, _HID), lambda m: (m, 0)),
            pl.BlockSpec((_HID, 128), lambda m: (0, 0)),
            pl.BlockSpec((1, 128), lambda m: (0, 0)),
        ],
        out_specs=pl.BlockSpec((m_blk, 128), lambda m: (m, 0)),
        out_shape=jax.ShapeDtypeStruct((2304, 128), jnp.float32),
    )(hidden, wh, bh)
    return logits


def _decode_body(cl_ref, o0_ref, o1_ref, o2_ref, o3_ref,
                 acx_ref, acy_ref, aw_ref, ah_ref,
                 s_out, y1_out, x1_out, y2_out, x2_out, sel_out):
    shape = (_ROWS, 128)
    riota = jax.lax.broadcasted_iota(jnp.int32, shape, 0)
    liota = jax.lax.broadcasted_iota(jnp.int32, shape, 1)
    fiota = riota * 128 + liota

    s = jax.nn.sigmoid(cl_ref[...])
    o2 = jnp.clip(o2_ref[...], -4.0, 4.0)
    o3 = jnp.clip(o3_ref[...], -4.0, 4.0)
    aw = aw_ref[...]
    ah = ah_ref[...]
    cx = acx_ref[...] + aw * o0_ref[...]
    cy = acy_ref[...] + ah * o1_ref[...]
    pw = aw * jnp.exp(o2)
    ph = ah * jnp.exp(o3)
    x1 = jnp.clip(cx - pw / 2, 0.0, _IMG)
    y1 = jnp.clip(cy - ph / 2, 0.0, _IMG)
    x2 = jnp.clip(cx + pw / 2, 0.0, _IMG)
    y2 = jnp.clip(cy + ph / 2, 0.0, _IMG)
    s_out[...] = s
    y1_out[...] = y1
    x1_out[...] = x1
    y2_out[...] = y2
    x2_out[...] = x2

    v = jax.lax.bitcast_convert_type(s, jnp.int32)
    v = jnp.where(fiota < _NA, v, jnp.int32(-1))

    def bis(_, lh):
        lo, hi = lh
        mid = lo + (hi - lo) // 2
        cnt = jnp.sum((v >= mid).astype(jnp.int32))
        big = cnt >= _PRE
        return (jnp.where(big, mid, lo), jnp.where(big, hi, mid))

    lo, hi = jax.lax.fori_loop(
        0, 32, bis, (jnp.int32(-2), jnp.int32(0x3F800001)))
    t = lo
    c_gt = jnp.sum((v > t).astype(jnp.int32))
    need = _PRE - c_gt

    def bis2(_, lh):
        lo2, hi2 = lh
        mid = lo2 + (hi2 - lo2) // 2
        cnt = jnp.sum(((v == t) & (fiota < mid)).astype(jnp.int32))
        enough = cnt >= need
        return (jnp.where(enough, lo2, mid), jnp.where(enough, mid, hi2))

    _, cut = jax.lax.fori_loop(
        0, 16, bis2, (jnp.int32(0), jnp.int32(_NPAD)))

    sel = (v > t) | ((v == t) & (fiota < cut))
    sel_out[...] = sel.astype(jnp.int32)


def _decode_call(cl, o0, o1, o2, o3, acx, acy, aw, ah):
    spec = pl.BlockSpec((_ROWS, 128), lambda: (0, 0))
    return pl.pallas_call(
        _decode_body,
        grid=(),
        in_specs=[spec] * 9,
        out_specs=[spec] * 6,
        out_shape=[jax.ShapeDtypeStruct((_ROWS, 128), jnp.float32)] * 5
        + [jax.ShapeDtypeStruct((_ROWS, 128), jnp.int32)],
    )(cl, o0, o1, o2, o3, acx, acy, aw, ah)


def _sc_compact(comb_hbm, pos_hbm, ccomb_hbm, sc_slice, posbuf, sem):
    wid = lax.axis_index("s")
    base = wid * _PERW
    pltpu.sync_copy(comb_hbm.at[pl.ds(base, _PERW)], sc_slice)
    pltpu.sync_copy(pos_hbm.at[pl.ds(base, _PERW)], posbuf)

    def scat(i, carry):
        pv = posbuf[pl.ds(i * 16, 16)]
        pltpu.async_copy(sc_slice.at[pl.ds(i * 16, 16)],
                         ccomb_hbm.at[pv], sem).wait()
        return carry

    jax.lax.fori_loop(0, _PERW // 16, scat, 0)


def _sc_compact_call(comb, pos):
    mesh = plsc.VectorSubcoreMesh(
        core_axis_name="c", subcore_axis_name="s", num_cores=1)
    kern = functools.partial(
        pl.kernel,
        mesh=mesh,
        compiler_params=pltpu.CompilerParams(use_tc_tiling_on_sc=False),
        out_type=jax.ShapeDtypeStruct((_CCAP, 16), jnp.float32),
        scratch_types=[
            pltpu.VMEM((_PERW, 16), jnp.float32),
            pltpu.VMEM((_PERW,), jnp.int32),
            pltpu.SemaphoreType.DMA,
        ],
    )(_sc_compact)
    return kern(comb, pos)


def _nms_body(cs_ref, cy1_ref, cx1_ref, cy2_ref, cx2_ref, cidx_ref, out_ref):
    shape = (_CROWS, 128)
    riota = jax.lax.broadcasted_iota(jnp.int32, shape, 0)
    liota = jax.lax.broadcasted_iota(jnp.int32, shape, 1)
    slot = riota * 128 + liota
    live = slot < _PRE

    ninf = jnp.float32(-jnp.inf)
    big = jnp.int32(_NPAD + 1)
    ms0 = jnp.where(live, cs_ref[...], ninf)
    orig = jnp.where(live, cidx_ref[...], big)

    y1 = cy1_ref[...]
    x1 = cx1_ref[...]
    y2 = cy2_ref[...]
    x2 = cx2_ref[...]
    areas = (y2 - y1) * (x2 - x1)

    out_ref[...] = jnp.zeros((304, 128), jnp.float32)
    l128 = jax.lax.broadcasted_iota(jnp.int32, (1, 128), 1)

    def step(i, ms):
        m = jnp.max(ms)
        valid = m > ninf
        eq = ms == m
        forig = jnp.min(jnp.where(eq, orig, big))
        pos = jnp.min(jnp.where(eq & (orig == forig), slot, jnp.int32(_CCAP)))
        r = pos // 128
        c = pos % 128

        def pick(ref):
            row = ref[pl.ds(r, 1), :]
            return jnp.sum(jnp.where(l128 == c, row, 0.0))

        by1 = pick(cy1_ref)
        bx1 = pick(cx1_ref)
        by2 = pick(cy2_ref)
        bx2 = pick(cx2_ref)
        ai = (by2 - by1) * (bx2 - bx1)
        yy1 = jnp.maximum(y1, by1)
        xx1 = jnp.maximum(x1, bx1)
        yy2 = jnp.minimum(y2, by2)
        xx2 = jnp.minimum(x2, bx2)
        inter = jnp.maximum(yy2 - yy1, 0.0) * jnp.maximum(xx2 - xx1, 0.0)
        iou = inter / (areas + ai - inter + 1e-9)
        ms = jnp.where((iou > _IOU) | (slot == pos), ninf, ms)
        vf = jnp.where(valid, 1.0, 0.0).astype(jnp.float32)
        row = (jnp.where(l128 == 0, by1, 0.0) + jnp.where(l128 == 1, bx1, 0.0)
               + jnp.where(l128 == 2, by2, 0.0)
               + jnp.where(l128 == 3, bx2, 0.0)) * vf
        out_ref[pl.ds(i, 1), :] = row
        return ms

    jax.lax.fori_loop(0, _POST, step, ms0)


def _nms_call(cs, cy1, cx1, cy2, cx2, cidx):
    spec = pl.BlockSpec((_CROWS, 128), lambda: (0, 0))
    return pl.pallas_call(
        _nms_body,
        grid=(),
        in_specs=[spec] * 6,
        out_specs=pl.BlockSpec((304, 128), lambda: (0, 0)),
        out_shape=jax.ShapeDtypeStruct((304, 128), jnp.float32),
    )(cs, cy1, cx1, cy2, cx2, cidx)


def _pad_plane(x, fill):
    return jnp.concatenate(
        [x, jnp.full((_NPAD - _NA,), fill, jnp.float32)]).reshape(_ROWS, 128)


def kernel(feature_map, conv1_w, conv1_b, cls_w, cls_b, reg_w, reg_b):
    fmp = jnp.pad(feature_map[0], ((1, 1), (1, 1), (0, 0)))
    parts = [fmp[dy:dy + _FH, dx:dx + _FW, :].reshape(_FH * _FW, _C)
             for dy in range(3) for dx in range(3)]
    x = jnp.concatenate(parts, axis=1)
    w = conv1_w.reshape(9 * _C, _HID)
    wh = jnp.zeros((_HID, 128), jnp.float32)
    wh = wh.at[:, :9].set(cls_w.reshape(_HID, 9))
    wh = wh.at[:, 9:45].set(reg_w.reshape(_HID, 36))
    bh = jnp.zeros((1, 128), jnp.float32)
    bh = bh.at[0, :9].set(cls_b)
    bh = bh.at[0, 9:45].set(reg_b)

    logits = _conv_heads(x, w, conv1_b.reshape(1, _HID), wh, bh)

    cl = _pad_plane(logits[:, :9].reshape(-1), -1e30)
    op = logits[:, 9:45].reshape(-1, 4)
    o0 = _pad_plane(op[:, 0], 0.0)
    o1 = _pad_plane(op[:, 1], 0.0)
    o2 = _pad_plane(op[:, 2], 0.0)
    o3 = _pad_plane(op[:, 3], 0.0)
    a = [jnp.asarray(p) for p in _APLANES]

    s_p, y1_p, x1_p, y2_p, x2_p, selarr = _decode_call(
        cl, o0, o1, o2, o3, a[0], a[1], a[2], a[3])

    sf = s_p.reshape(-1)
    idxf = jax.lax.bitcast_convert_type(
        jnp.arange(_NPAD, dtype=jnp.int32), jnp.float32)
    zerop = jnp.zeros((_NPAD,), jnp.float32)
    comb = jnp.stack(
        [sf, y1_p.reshape(-1), x1_p.reshape(-1), y2_p.reshape(-1),
         x2_p.reshape(-1), idxf] + [zerop] * 10, axis=1)
    selm = selarr.reshape(-1)
    rank = jnp.cumsum(selm) - selm
    gidx = jnp.arange(_NPAD, dtype=jnp.int32)
    pos = jnp.where(selm > 0, rank, 2048 + gidx // _PERW).astype(jnp.int32)

    ccomb = _sc_compact_call(comb, pos)

    cs = ccomb[:, 0].reshape(_CROWS, 128)
    cy1 = ccomb[:, 1].reshape(_CROWS, 128)
    cx1 = ccomb[:, 2].reshape(_CROWS, 128)
    cy2 = ccomb[:, 3].reshape(_CROWS, 128)
    cx2 = ccomb[:, 4].reshape(_CROWS, 128)
    cidx = jax.lax.bitcast_convert_type(
        ccomb[:, 5], jnp.int32).reshape(_CROWS, 128)

    out = _nms_call(cs, cy1, cx1, cy2, cx2, cidx)
    return out[:_POST, :4]
